# Initial kernel scaffold; baseline (speedup 1.0000x reference)
import jax
import jax.numpy as jnp
from jax.experimental import pallas as pl


def kernel(x, edge_index, W1, b1, W2, b2, Wg1, bg1, Wg2, bg2):
    raise NotImplementedError("write your pallas kernel here")


# speedup vs baseline: 5.7037x; 5.7037x over previous
import functools

import jax
import jax.numpy as jnp
from jax import lax
from jax.experimental import pallas as pl
from jax.experimental.pallas import tpu as pltpu
from jax.experimental.pallas import tpu_sc as plsc

N = 10000
E = 640000
IN_DIM = 128
HID = 256
RNI_DIM = 224
OUT2 = 32

NT = 16
NW = 2 * NT
CHUNK = 80
SEGC = 64
EPAD = 655360
NSEG = EPAD // (NT * SEGC * CHUNK)
NSEG = EPAD // NT // (SEGC * CHUNK)
NSEG2 = EPAD // NW // (SEGC * CHUNK)
NPAD = 10240
ROWS_PT = NPAD // NT

@functools.cache
def _make_agg_feat():
    @functools.partial(
        pl.kernel,
        out_type=jax.ShapeDtypeStruct((2 * NPAD, 128), jnp.float32),
        mesh=plsc.VectorSubcoreMesh(core_axis_name="c", subcore_axis_name="s"),
        scratch_types=[
            pltpu.VMEM((SEGC * CHUNK,), jnp.int32),
            pltpu.VMEM((SEGC, CHUNK), jnp.int32),
            pltpu.VMEM((CHUNK, 128), jnp.float32),
            pltpu.VMEM_SHARED((NPAD, 128), jnp.float32),
            pltpu.SemaphoreType.DMA,
        ],
    )
    def agg(hs2, zeros_hbm, src2_hbm, dst_hbm, out, src_v, dst_v, rows_v,
            acc, sem):
        c = lax.axis_index("c")
        s = lax.axis_index("s")
        pltpu.sync_copy(zeros_hbm.at[pl.ds(s * ROWS_PT, ROWS_PT)],
                        acc.at[pl.ds(s * ROWS_PT, ROWS_PT)])
        plsc.subcore_barrier()

        def chunk_body(j, carry):
            pltpu.async_copy(hs2.at[src_v.at[pl.ds(j * CHUNK, CHUNK)]],
                             rows_v, sem).wait()
            pltpu.sync_copy(rows_v, acc.at[dst_v.at[j]], add=True)
            return carry

        for g in range(NSEG):
            pltpu.sync_copy(src2_hbm.at[(c * NT + s) * NSEG + g], src_v)
            pltpu.sync_copy(dst_hbm.at[s * NSEG + g], dst_v)
            lax.fori_loop(0, SEGC, chunk_body, 0)

        plsc.subcore_barrier()
        pltpu.sync_copy(acc.at[pl.ds(s * ROWS_PT, ROWS_PT)],
                        out.at[pl.ds(c * NPAD + s * ROWS_PT, ROWS_PT)])

    return agg


@functools.cache
def _make_agg_edge(ones_mode):
    @functools.partial(
        pl.kernel,
        out_type=jax.ShapeDtypeStruct((2 * NPAD, 128), jnp.float32),
        mesh=plsc.VectorSubcoreMesh(core_axis_name="c", subcore_axis_name="s"),
        scratch_types=[
            pltpu.VMEM((SEGC * CHUNK,), jnp.int32),
            pltpu.VMEM((SEGC, CHUNK), jnp.int32),
            pltpu.VMEM((CHUNK, 128), jnp.float32),
            pltpu.VMEM_SHARED((NPAD, 128), jnp.float32),
            pltpu.SemaphoreType.DMA,
        ],
    )
    def agg(*refs):
        if ones_mode:
            (ones_hbm, zeros_hbm, dst_hbm, out,
             src_v, dst_v, rows_v, acc, sem) = refs
            hs = None
        else:
            (hs, zeros_hbm, src_hbm, dst_hbm, out,
             src_v, dst_v, rows_v, acc, sem) = refs
        c = lax.axis_index("c")
        s = lax.axis_index("s")
        w = c * NT + s
        pltpu.sync_copy(zeros_hbm.at[pl.ds(s * ROWS_PT, ROWS_PT)],
                        acc.at[pl.ds(s * ROWS_PT, ROWS_PT)])
        if ones_mode:
            pltpu.sync_copy(ones_hbm, rows_v)
        plsc.subcore_barrier()

        def chunk_body(j, carry):
            if not ones_mode:
                pltpu.async_copy(hs.at[src_v.at[pl.ds(j * CHUNK, CHUNK)]],
                                 rows_v, sem).wait()
            pltpu.sync_copy(rows_v, acc.at[dst_v.at[j]], add=True)
            return carry

        for g in range(NSEG2):
            if not ones_mode:
                pltpu.sync_copy(src_hbm.at[w * NSEG2 + g], src_v)
            pltpu.sync_copy(dst_hbm.at[w * NSEG2 + g], dst_v)
            lax.fori_loop(0, SEGC, chunk_body, 0)

        plsc.subcore_barrier()
        pltpu.sync_copy(acc.at[pl.ds(s * ROWS_PT, ROWS_PT)],
                        out.at[pl.ds(c * NPAD + s * ROWS_PT, ROWS_PT)])

    return agg


def _elu(x):
    return jnp.where(x > 0, x, jnp.exp(jnp.minimum(x, 0.0)) - 1.0)


_RB = 1000
_GRID = (N // _RB,)
_DOT = dict(preferred_element_type=jnp.float32,
            precision=jax.lax.Precision.HIGHEST)


def _rows(width):
    return pl.BlockSpec((_RB, width), lambda i: (i, 0))


def _full(shape):
    return pl.BlockSpec(shape, lambda i: (0, 0))


def _tc_a(x, w1, dinv, r, wg1b):
    def body(x_ref, w1_ref, dinv_ref, r_ref, wg1b_ref,
             p1_ref, s0_ref, s1_ref, rp_ref):
        p = jnp.dot(x_ref[...], w1_ref[...], **_DOT)
        p1_ref[...] = p
        sv = p * dinv_ref[...]
        s0_ref[...] = sv[:, :128]
        s1_ref[...] = sv[:, 128:]
        rp_ref[...] = jnp.dot(r_ref[...], wg1b_ref[...], **_DOT)

    return pl.pallas_call(
        body,
        grid=_GRID,
        in_specs=[_rows(IN_DIM), _full((IN_DIM, HID)), _rows(1),
                  _rows(RNI_DIM), _full((RNI_DIM, HID))],
        out_specs=[_rows(HID), _rows(128), _rows(128), _rows(HID)],
        out_shape=[jax.ShapeDtypeStruct((N, HID), jnp.float32),
                   jax.ShapeDtypeStruct((N, 128), jnp.float32),
                   jax.ShapeDtypeStruct((N, 128), jnp.float32),
                   jax.ShapeDtypeStruct((N, HID), jnp.float32)],
    )(x, w1, dinv, r, wg1b)


def _tc_b(a0, a1, p1, b1, dinv, w2):
    def body(a0_ref, a1_ref, p1_ref, b1_ref, dinv_ref, w2_ref,
             p2_ref, sp_ref):
        dinv = dinv_ref[...]
        a = jnp.concatenate([a0_ref[...], a1_ref[...]], axis=1)
        h = _elu(dinv * a + (dinv * dinv) * p1_ref[...] + b1_ref[...])
        p2 = jnp.dot(h, w2_ref[...], **_DOT)
        p2_ref[...] = p2
        sv = p2 * dinv
        sp_ref[...] = jnp.concatenate(
            [sv, jnp.zeros((_RB, 128 - OUT2), jnp.float32)], axis=1)

    return pl.pallas_call(
        body,
        grid=_GRID,
        in_specs=[_rows(128), _rows(128), _rows(HID), _full((1, HID)),
                  _rows(1), _full((HID, OUT2))],
        out_specs=[_rows(OUT2), _rows(128)],
        out_shape=[jax.ShapeDtypeStruct((N, OUT2), jnp.float32),
                   jax.ShapeDtypeStruct((N, 128), jnp.float32)],
    )(a0, a1, p1, b1, dinv, w2)


def _tc_c(a0, a1, p2, b2, dinv, wg1a, rproj):
    def body(a0_ref, a1_ref, p2_ref, b2_ref, dinv_ref, wg1a_ref, rp_ref,
             p3_ref, s0_ref, s1_ref):
        dinv = dinv_ref[...]
        a = a0_ref[...] + a1_ref[...]
        h = _elu(dinv * a + (dinv * dinv) * p2_ref[...] + b2_ref[...])
        p3 = jnp.dot(h, wg1a_ref[...], **_DOT) + rp_ref[...]
        p3_ref[...] = p3
        sv = p3 * dinv
        s0_ref[...] = sv[:, :128]
        s1_ref[...] = sv[:, 128:]

    return pl.pallas_call(
        body,
        grid=_GRID,
        in_specs=[_rows(OUT2), _rows(OUT2), _rows(OUT2), _full((1, OUT2)),
                  _rows(1), _full((OUT2, HID)), _rows(HID)],
        out_specs=[_rows(HID), _rows(128), _rows(128)],
        out_shape=[jax.ShapeDtypeStruct((N, HID), jnp.float32),
                   jax.ShapeDtypeStruct((N, 128), jnp.float32),
                   jax.ShapeDtypeStruct((N, 128), jnp.float32)],
    )(a0, a1, p2, b2, dinv, wg1a, rproj)


def _tc_d(a0, a1, p3, bg1, dinv, wg2):
    def body(a0_ref, a1_ref, p3_ref, bg1_ref, dinv_ref, wg2_ref,
             p4_ref, s0_ref, s1_ref):
        dinv = dinv_ref[...]
        a = jnp.concatenate([a0_ref[...], a1_ref[...]], axis=1)
        h = _elu(dinv * a + (dinv * dinv) * p3_ref[...] + bg1_ref[...])
        p4 = jnp.dot(h, wg2_ref[...], **_DOT)
        p4_ref[...] = p4
        sv = p4 * dinv
        s0_ref[...] = sv[:, :128]
        s1_ref[...] = sv[:, 128:]

    return pl.pallas_call(
        body,
        grid=_GRID,
        in_specs=[_rows(128), _rows(128), _rows(HID), _full((1, HID)),
                  _rows(1), _full((HID, HID))],
        out_specs=[_rows(HID), _rows(128), _rows(128)],
        out_shape=[jax.ShapeDtypeStruct((N, HID), jnp.float32),
                   jax.ShapeDtypeStruct((N, 128), jnp.float32),
                   jax.ShapeDtypeStruct((N, 128), jnp.float32)],
    )(a0, a1, p3, bg1, dinv, wg2)


def _tc_e(a0, a1, p4, bg2, dinv):
    def body(a0_ref, a1_ref, p4_ref, bg2_ref, dinv_ref, out_ref):
        dinv = dinv_ref[...]
        a = jnp.concatenate([a0_ref[...], a1_ref[...]], axis=1)
        out_ref[...] = dinv * a + (dinv * dinv) * p4_ref[...] + bg2_ref[...]

    return pl.pallas_call(
        body,
        grid=_GRID,
        in_specs=[_rows(128), _rows(128), _rows(HID), _full((1, HID)),
                  _rows(1)],
        out_specs=_rows(HID),
        out_shape=jax.ShapeDtypeStruct((N, HID), jnp.float32),
    )(a0, a1, p4, bg2, dinv)


def kernel(x, edge_index, W1, b1, W2, b2, Wg1, bg1, Wg2, bg2):
    src = edge_index[0].astype(jnp.int32)
    dst = edge_index[1].astype(jnp.int32)
    trash = jnp.int32(NPAD - 1)
    src_p = jnp.full((EPAD,), trash, jnp.int32).at[:E].set(src)
    dst_p = jnp.full((EPAD,), trash, jnp.int32).at[:E].set(dst)

    src2 = jnp.concatenate([src_p, src_p + NPAD]).reshape(
        2 * NT * NSEG, SEGC * CHUNK)
    dst16 = dst_p.reshape(NT * NSEG, SEGC, CHUNK)
    srcB = src_p.reshape(NW * NSEG2, SEGC * CHUNK)
    dstB = dst_p.reshape(NW * NSEG2, SEGC, CHUNK)

    zeros128 = jnp.zeros((NPAD, 128), jnp.float32)
    ones_row = jnp.ones((CHUNK, 128), jnp.float32)

    _agg128 = _make_agg_feat()
    _agg_narrow = _make_agg_edge(False)
    _agg_deg = _make_agg_edge(True)

    def _stack2(a0, a1):
        h = jnp.zeros((2 * NPAD, 128), jnp.float32)
        return h.at[:N].set(a0).at[NPAD:NPAD + N].set(a1)

    def _pad1(a):
        return jnp.pad(a, ((0, NPAD - N), (0, 0)))

    dsum = _agg_deg(ones_row, zeros128, dstB)
    deg = dsum[:N, 0] + dsum[NPAD:NPAD + N, 0] + 1.0
    dinv = lax.rsqrt(deg)[:, None]

    r = jax.random.normal(jax.random.key(42), (N, RNI_DIM), dtype=jnp.float32)

    b1r = b1[None, :]
    b2r = b2[None, :]
    bg1r = bg1[None, :]
    bg2r = bg2[None, :]

    p1, s10, s11, rproj = _tc_a(x, W1, dinv, r, Wg1[OUT2:])
    a1 = _agg128(_stack2(s10, s11), zeros128, src2, dst16)
    p2, s2p = _tc_b(a1[:N], a1[NPAD:NPAD + N], p1, b1r, dinv, W2)
    a2 = _agg_narrow(_pad1(s2p), zeros128, srcB, dstB)
    p3, s30, s31 = _tc_c(a2[:N, :OUT2], a2[NPAD:NPAD + N, :OUT2], p2, b2r,
                         dinv, Wg1[:OUT2], rproj)
    a3 = _agg128(_stack2(s30, s31), zeros128, src2, dst16)
    p4, s40, s41 = _tc_d(a3[:N], a3[NPAD:NPAD + N], p3, bg1r, dinv, Wg2)
    a4 = _agg128(_stack2(s40, s41), zeros128, src2, dst16)
    return _tc_e(a4[:N], a4[NPAD:NPAD + N], p4, bg2r, dinv)


# speedup vs baseline: 7.0592x; 1.2376x over previous
import functools

import jax
import jax.numpy as jnp
from jax import lax
from jax.experimental import pallas as pl
from jax.experimental.pallas import tpu as pltpu
from jax.experimental.pallas import tpu_sc as plsc

N = 10000
E = 640000
IN_DIM = 128
HID = 256
RNI_DIM = 224
OUT2 = 32

NT = 16
NW = 2 * NT
CHUNK = 80
SEGC = 64
EPAD = 655360
NSEG = EPAD // (NT * SEGC * CHUNK)
NSEG = EPAD // NT // (SEGC * CHUNK)
NSEG2 = EPAD // NW // (SEGC * CHUNK)
NPAD = 10240
ROWS_PT = NPAD // NT

@functools.cache
def _make_agg_feat():
    @functools.partial(
        pl.kernel,
        out_type=jax.ShapeDtypeStruct((2 * NPAD, 128), jnp.float32),
        mesh=plsc.VectorSubcoreMesh(core_axis_name="c", subcore_axis_name="s"),
        scratch_types=[
            pltpu.VMEM((SEGC * CHUNK,), jnp.int32),
            pltpu.VMEM((SEGC, CHUNK), jnp.int32),
            pltpu.VMEM((CHUNK, 128), jnp.float32),
            pltpu.VMEM((CHUNK, 128), jnp.float32),
            pltpu.VMEM_SHARED((NPAD, 128), jnp.float32),
            pltpu.SemaphoreType.DMA,
            pltpu.SemaphoreType.DMA,
        ],
    )
    def agg(hs2, zeros_hbm, src2_hbm, dst_hbm, out, src_v, dst_v, rows_a,
            rows_b, acc, sem_a, sem_b):
        c = lax.axis_index("c")
        s = lax.axis_index("s")
        pltpu.sync_copy(zeros_hbm.at[pl.ds(s * ROWS_PT, ROWS_PT)],
                        acc.at[pl.ds(s * ROWS_PT, ROWS_PT)])
        plsc.subcore_barrier()

        def _gather(j, buf, sem):
            pltpu.async_copy(hs2.at[src_v.at[pl.ds(j * CHUNK, CHUNK)]],
                             buf, sem)

        def _wait(buf, sem):
            pltpu.make_async_copy(hs2.at[src_v.at[pl.ds(0, CHUNK)]],
                                  buf, sem).wait()

        def pipe_body(k, carry):
            j = 2 * k
            _gather(j + 1, rows_b, sem_b)
            _wait(rows_a, sem_a)
            pltpu.sync_copy(rows_a, acc.at[dst_v.at[j]], add=True)
            _gather((j + 2) & (SEGC - 1), rows_a, sem_a)
            _wait(rows_b, sem_b)
            pltpu.sync_copy(rows_b, acc.at[dst_v.at[j + 1]], add=True)
            return carry

        for g in range(NSEG):
            pltpu.sync_copy(src2_hbm.at[(c * NT + s) * NSEG + g], src_v)
            pltpu.sync_copy(dst_hbm.at[s * NSEG + g], dst_v)
            _gather(0, rows_a, sem_a)
            lax.fori_loop(0, SEGC // 2, pipe_body, 0)
            _wait(rows_a, sem_a)

        plsc.subcore_barrier()
        pltpu.sync_copy(acc.at[pl.ds(s * ROWS_PT, ROWS_PT)],
                        out.at[pl.ds(c * NPAD + s * ROWS_PT, ROWS_PT)])

    return agg


@functools.cache
def _make_agg_edge(ones_mode):
    @functools.partial(
        pl.kernel,
        out_type=jax.ShapeDtypeStruct((2 * NPAD, 128), jnp.float32),
        mesh=plsc.VectorSubcoreMesh(core_axis_name="c", subcore_axis_name="s"),
        scratch_types=[
            pltpu.VMEM((SEGC * CHUNK,), jnp.int32),
            pltpu.VMEM((SEGC, CHUNK), jnp.int32),
            pltpu.VMEM((CHUNK, 128), jnp.float32),
            pltpu.VMEM((CHUNK, 128), jnp.float32),
            pltpu.VMEM_SHARED((NPAD, 128), jnp.float32),
            pltpu.SemaphoreType.DMA,
            pltpu.SemaphoreType.DMA,
        ],
    )
    def agg(*refs):
        if ones_mode:
            (ones_hbm, zeros_hbm, dst_hbm, out,
             src_v, dst_v, rows_a, rows_b, acc, sem_a, sem_b) = refs
            hs = None
        else:
            (hs, zeros_hbm, src_hbm, dst_hbm, out,
             src_v, dst_v, rows_a, rows_b, acc, sem_a, sem_b) = refs
        c = lax.axis_index("c")
        s = lax.axis_index("s")
        w = c * NT + s
        pltpu.sync_copy(zeros_hbm.at[pl.ds(s * ROWS_PT, ROWS_PT)],
                        acc.at[pl.ds(s * ROWS_PT, ROWS_PT)])
        if ones_mode:
            pltpu.sync_copy(ones_hbm, rows_a)
        plsc.subcore_barrier()

        if ones_mode:
            def chunk_body(j, carry):
                pltpu.sync_copy(rows_a, acc.at[dst_v.at[j]], add=True)
                return carry

            for g in range(NSEG2):
                pltpu.sync_copy(dst_hbm.at[w * NSEG2 + g], dst_v)
                lax.fori_loop(0, SEGC, chunk_body, 0)
        else:
            def _gather(j, buf, sem):
                pltpu.async_copy(hs.at[src_v.at[pl.ds(j * CHUNK, CHUNK)]],
                                 buf, sem)

            def _wait(buf, sem):
                pltpu.make_async_copy(hs.at[src_v.at[pl.ds(0, CHUNK)]],
                                      buf, sem).wait()

            def pipe_body(k, carry):
                j = 2 * k
                _gather(j + 1, rows_b, sem_b)
                _wait(rows_a, sem_a)
                pltpu.sync_copy(rows_a, acc.at[dst_v.at[j]], add=True)
                _gather((j + 2) & (SEGC - 1), rows_a, sem_a)
                _wait(rows_b, sem_b)
                pltpu.sync_copy(rows_b, acc.at[dst_v.at[j + 1]], add=True)
                return carry

            for g in range(NSEG2):
                pltpu.sync_copy(src_hbm.at[w * NSEG2 + g], src_v)
                pltpu.sync_copy(dst_hbm.at[w * NSEG2 + g], dst_v)
                _gather(0, rows_a, sem_a)
                lax.fori_loop(0, SEGC // 2, pipe_body, 0)
                _wait(rows_a, sem_a)

        plsc.subcore_barrier()
        pltpu.sync_copy(acc.at[pl.ds(s * ROWS_PT, ROWS_PT)],
                        out.at[pl.ds(c * NPAD + s * ROWS_PT, ROWS_PT)])

    return agg


def _elu(x):
    return jnp.where(x > 0, x, jnp.exp(jnp.minimum(x, 0.0)) - 1.0)


_RB = 1000
_GRID = (N // _RB,)
_DOT = dict(preferred_element_type=jnp.float32,
            precision=jax.lax.Precision.HIGHEST)


def _rows(width):
    return pl.BlockSpec((_RB, width), lambda i: (i, 0))


def _full(shape):
    return pl.BlockSpec(shape, lambda i: (0, 0))


def _tc_a(x, w1, dinv, r, wg1b):
    def body(x_ref, w1_ref, dinv_ref, r_ref, wg1b_ref,
             p1_ref, s0_ref, s1_ref, rp_ref):
        p = jnp.dot(x_ref[...], w1_ref[...], **_DOT)
        p1_ref[...] = p
        sv = p * dinv_ref[...]
        s0_ref[...] = sv[:, :128]
        s1_ref[...] = sv[:, 128:]
        rp_ref[...] = jnp.dot(r_ref[...], wg1b_ref[...], **_DOT)

    return pl.pallas_call(
        body,
        grid=_GRID,
        in_specs=[_rows(IN_DIM), _full((IN_DIM, HID)), _rows(1),
                  _rows(RNI_DIM), _full((RNI_DIM, HID))],
        out_specs=[_rows(HID), _rows(128), _rows(128), _rows(HID)],
        out_shape=[jax.ShapeDtypeStruct((N, HID), jnp.float32),
                   jax.ShapeDtypeStruct((N, 128), jnp.float32),
                   jax.ShapeDtypeStruct((N, 128), jnp.float32),
                   jax.ShapeDtypeStruct((N, HID), jnp.float32)],
    )(x, w1, dinv, r, wg1b)


def _tc_b(a0, a1, p1, b1, dinv, w2):
    def body(a0_ref, a1_ref, p1_ref, b1_ref, dinv_ref, w2_ref,
             p2_ref, sp_ref):
        dinv = dinv_ref[...]
        a = jnp.concatenate([a0_ref[...], a1_ref[...]], axis=1)
        h = _elu(dinv * a + (dinv * dinv) * p1_ref[...] + b1_ref[...])
        p2 = jnp.dot(h, w2_ref[...], **_DOT)
        p2_ref[...] = p2
        sv = p2 * dinv
        sp_ref[...] = jnp.concatenate(
            [sv, jnp.zeros((_RB, 128 - OUT2), jnp.float32)], axis=1)

    return pl.pallas_call(
        body,
        grid=_GRID,
        in_specs=[_rows(128), _rows(128), _rows(HID), _full((1, HID)),
                  _rows(1), _full((HID, OUT2))],
        out_specs=[_rows(OUT2), _rows(128)],
        out_shape=[jax.ShapeDtypeStruct((N, OUT2), jnp.float32),
                   jax.ShapeDtypeStruct((N, 128), jnp.float32)],
    )(a0, a1, p1, b1, dinv, w2)


def _tc_c(a0, a1, p2, b2, dinv, wg1a, rproj):
    def body(a0_ref, a1_ref, p2_ref, b2_ref, dinv_ref, wg1a_ref, rp_ref,
             p3_ref, s0_ref, s1_ref):
        dinv = dinv_ref[...]
        a = a0_ref[...] + a1_ref[...]
        h = _elu(dinv * a + (dinv * dinv) * p2_ref[...] + b2_ref[...])
        p3 = jnp.dot(h, wg1a_ref[...], **_DOT) + rp_ref[...]
        p3_ref[...] = p3
        sv = p3 * dinv
        s0_ref[...] = sv[:, :128]
        s1_ref[...] = sv[:, 128:]

    return pl.pallas_call(
        body,
        grid=_GRID,
        in_specs=[_rows(OUT2), _rows(OUT2), _rows(OUT2), _full((1, OUT2)),
                  _rows(1), _full((OUT2, HID)), _rows(HID)],
        out_specs=[_rows(HID), _rows(128), _rows(128)],
        out_shape=[jax.ShapeDtypeStruct((N, HID), jnp.float32),
                   jax.ShapeDtypeStruct((N, 128), jnp.float32),
                   jax.ShapeDtypeStruct((N, 128), jnp.float32)],
    )(a0, a1, p2, b2, dinv, wg1a, rproj)


def _tc_d(a0, a1, p3, bg1, dinv, wg2):
    def body(a0_ref, a1_ref, p3_ref, bg1_ref, dinv_ref, wg2_ref,
             p4_ref, s0_ref, s1_ref):
        dinv = dinv_ref[...]
        a = jnp.concatenate([a0_ref[...], a1_ref[...]], axis=1)
        h = _elu(dinv * a + (dinv * dinv) * p3_ref[...] + bg1_ref[...])
        p4 = jnp.dot(h, wg2_ref[...], **_DOT)
        p4_ref[...] = p4
        sv = p4 * dinv
        s0_ref[...] = sv[:, :128]
        s1_ref[...] = sv[:, 128:]

    return pl.pallas_call(
        body,
        grid=_GRID,
        in_specs=[_rows(128), _rows(128), _rows(HID), _full((1, HID)),
                  _rows(1), _full((HID, HID))],
        out_specs=[_rows(HID), _rows(128), _rows(128)],
        out_shape=[jax.ShapeDtypeStruct((N, HID), jnp.float32),
                   jax.ShapeDtypeStruct((N, 128), jnp.float32),
                   jax.ShapeDtypeStruct((N, 128), jnp.float32)],
    )(a0, a1, p3, bg1, dinv, wg2)


def _tc_e(a0, a1, p4, bg2, dinv):
    def body(a0_ref, a1_ref, p4_ref, bg2_ref, dinv_ref, out_ref):
        dinv = dinv_ref[...]
        a = jnp.concatenate([a0_ref[...], a1_ref[...]], axis=1)
        out_ref[...] = dinv * a + (dinv * dinv) * p4_ref[...] + bg2_ref[...]

    return pl.pallas_call(
        body,
        grid=_GRID,
        in_specs=[_rows(128), _rows(128), _rows(HID), _full((1, HID)),
                  _rows(1)],
        out_specs=_rows(HID),
        out_shape=jax.ShapeDtypeStruct((N, HID), jnp.float32),
    )(a0, a1, p4, bg2, dinv)


def kernel(x, edge_index, W1, b1, W2, b2, Wg1, bg1, Wg2, bg2):
    src = edge_index[0].astype(jnp.int32)
    dst = edge_index[1].astype(jnp.int32)
    trash = jnp.int32(NPAD - 1)
    src_p = jnp.full((EPAD,), trash, jnp.int32).at[:E].set(src)
    dst_p = jnp.full((EPAD,), trash, jnp.int32).at[:E].set(dst)

    src2 = jnp.concatenate([src_p, src_p + NPAD]).reshape(
        2 * NT * NSEG, SEGC * CHUNK)
    dst16 = dst_p.reshape(NT * NSEG, SEGC, CHUNK)
    srcB = src_p.reshape(NW * NSEG2, SEGC * CHUNK)
    dstB = dst_p.reshape(NW * NSEG2, SEGC, CHUNK)

    zeros128 = jnp.zeros((NPAD, 128), jnp.float32)
    ones_row = jnp.ones((CHUNK, 128), jnp.float32)

    _agg128 = _make_agg_feat()
    _agg_narrow = _make_agg_edge(False)
    _agg_deg = _make_agg_edge(True)

    def _stack2(a0, a1):
        h = jnp.zeros((2 * NPAD, 128), jnp.float32)
        return h.at[:N].set(a0).at[NPAD:NPAD + N].set(a1)

    def _pad1(a):
        return jnp.pad(a, ((0, NPAD - N), (0, 0)))

    dsum = _agg_deg(ones_row, zeros128, dstB)
    deg = dsum[:N, 0] + dsum[NPAD:NPAD + N, 0] + 1.0
    dinv = lax.rsqrt(deg)[:, None]

    r = jax.random.normal(jax.random.key(42), (N, RNI_DIM), dtype=jnp.float32)

    b1r = b1[None, :]
    b2r = b2[None, :]
    bg1r = bg1[None, :]
    bg2r = bg2[None, :]

    p1, s10, s11, rproj = _tc_a(x, W1, dinv, r, Wg1[OUT2:])
    a1 = _agg128(_stack2(s10, s11), zeros128, src2, dst16)
    p2, s2p = _tc_b(a1[:N], a1[NPAD:NPAD + N], p1, b1r, dinv, W2)
    a2 = _agg_narrow(_pad1(s2p), zeros128, srcB, dstB)
    p3, s30, s31 = _tc_c(a2[:N, :OUT2], a2[NPAD:NPAD + N, :OUT2], p2, b2r,
                         dinv, Wg1[:OUT2], rproj)
    a3 = _agg128(_stack2(s30, s31), zeros128, src2, dst16)
    p4, s40, s41 = _tc_d(a3[:N], a3[NPAD:NPAD + N], p3, bg1r, dinv, Wg2)
    a4 = _agg128(_stack2(s40, s41), zeros128, src2, dst16)
    return _tc_e(a4[:N], a4[NPAD:NPAD + N], p4, bg2r, dinv)
